# baseline (speedup 1.0000x reference)
import functools

import jax
import jax.numpy as jnp
from jax import lax
from jax.experimental import pallas as pl
from jax.experimental.pallas import tpu as pltpu
from jax.experimental.pallas import tpu_sc as plsc

N = 10000
E = 320000
F = 128
C = 64

NC = 2
NS = 16
NW = NC * NS
CHUNK = 512
NCHUNKS = E // CHUNK
CH_BASE = NCHUNKS // NW
CH_EXTRA = NCHUNKS % NW
N_PAD = 10240
ROWS_PER_TILE = N_PAD // NS
INIT_SLICES = []
_off = 0
while _off < ROWS_PER_TILE:
    _sz = min(CHUNK, ROWS_PER_TILE - _off)
    INIT_SLICES.append((_off, _sz))
    _off += _sz
ZCHUNK = 128


def _sc_mesh():
    return plsc.VectorSubcoreMesh(core_axis_name="c", subcore_axis_name="s")


_SC_PARAMS = pltpu.CompilerParams(use_tc_tiling_on_sc=False)


def _chunk_range(wid):
    start = CH_BASE * wid + lax.min(wid, CH_EXTRA)
    return start


def _sc_degree(dst2):
    @functools.partial(
        pl.kernel,
        out_type=jax.ShapeDtypeStruct((NC, N_PAD), jnp.float32),
        mesh=_sc_mesh(),
        compiler_params=_SC_PARAMS,
        scratch_types=[
            pltpu.VMEM((CH_BASE + 1, CHUNK), jnp.int32),
            pltpu.VMEM((CHUNK,), jnp.float32),
            pltpu.VMEM((ROWS_PER_TILE,), jnp.float32),
            pltpu.VMEM_SHARED((N_PAD,), jnp.float32),
        ],
    )
    def deg_kernel(ei_hbm, out_hbm, dst_v, ones_v, buf_v, deg_sh):
        cid = lax.axis_index("c")
        sid = lax.axis_index("s")
        wid = cid * NS + sid
        start = _chunk_range(wid)

        def fill(i, _):
            buf_v[pl.ds(i * 16, 16)] = jnp.zeros((16,), jnp.float32)
            return 0

        lax.fori_loop(0, ROWS_PER_TILE // 16, fill, 0)

        def fill1(i, _):
            ones_v[pl.ds(i * 16, 16)] = jnp.ones((16,), jnp.float32)
            return 0

        lax.fori_loop(0, CHUNK // 16, fill1, 0)

        tile_rows = pl.ds(sid * ROWS_PER_TILE, ROWS_PER_TILE)
        pltpu.sync_copy(buf_v, deg_sh.at[tile_rows])
        pltpu.sync_copy(ei_hbm.at[pl.ds(start, CH_BASE)],
                        dst_v.at[pl.ds(0, CH_BASE)])

        @pl.when(wid < CH_EXTRA)
        def _():
            pltpu.sync_copy(ei_hbm.at[pl.ds(start + CH_BASE, 1)],
                            dst_v.at[pl.ds(CH_BASE, 1)])

        plsc.subcore_barrier()

        def body(j, _):
            pltpu.sync_copy(ones_v, deg_sh.at[dst_v.at[j]], add=True)
            return 0

        lax.fori_loop(0, CH_BASE, body, 0)

        @pl.when(wid < CH_EXTRA)
        def _():
            pltpu.sync_copy(ones_v, deg_sh.at[dst_v.at[CH_BASE]], add=True)

        plsc.subcore_barrier()
        pltpu.sync_copy(deg_sh.at[tile_rows], buf_v)
        pltpu.sync_copy(buf_v, out_hbm.at[cid, tile_rows])

    return deg_kernel(dst2)


def _sc_aggregate(y_ext, src2, dst2):
    @functools.partial(
        pl.kernel,
        out_type=jax.ShapeDtypeStruct((NC, N_PAD, C), jnp.float32),
        mesh=_sc_mesh(),
        compiler_params=_SC_PARAMS,
        scratch_types=[
            pltpu.VMEM((CH_BASE + 1, CHUNK), jnp.int32),
            pltpu.VMEM((CH_BASE + 1, CHUNK), jnp.int32),
            pltpu.VMEM((CHUNK, C), jnp.float32),
            pltpu.VMEM((CHUNK, C), jnp.float32),
            pltpu.VMEM_SHARED((N_PAD, C), jnp.float32),
            pltpu.SemaphoreType.DMA,
            pltpu.SemaphoreType.DMA,
        ],
    )
    def agg_kernel(y_hbm, src_hbm, dst_hbm, out_hbm, src_v, dst_v, rows_a,
                   rows_b, agg_sh, semg_a, semg_b):
        cid = lax.axis_index("c")
        sid = lax.axis_index("s")
        wid = cid * NS + sid
        start = _chunk_range(wid)

        def zero_row(i, _):
            for k in range(C // 16):
                rows_b[i, pl.ds(k * 16, 16)] = jnp.zeros((16,), jnp.float32)
            return 0

        lax.fori_loop(0, CHUNK, zero_row, 0)

        for off, sz in INIT_SLICES:
            sl = pl.ds(sid * ROWS_PER_TILE + off, sz)

            @pl.when(cid == 0)
            def _():
                pltpu.sync_copy(y_hbm.at[sl], rows_a.at[pl.ds(0, sz)])
                pltpu.sync_copy(rows_a.at[pl.ds(0, sz)], agg_sh.at[sl])

            @pl.when(cid != 0)
            def _():
                pltpu.sync_copy(rows_b.at[pl.ds(0, sz)], agg_sh.at[sl])

        pltpu.sync_copy(src_hbm.at[pl.ds(start, CH_BASE)],
                        src_v.at[pl.ds(0, CH_BASE)])
        pltpu.sync_copy(dst_hbm.at[pl.ds(start, CH_BASE)],
                        dst_v.at[pl.ds(0, CH_BASE)])

        @pl.when(wid < CH_EXTRA)
        def _():
            pltpu.sync_copy(src_hbm.at[pl.ds(start + CH_BASE, 1)],
                            src_v.at[pl.ds(CH_BASE, 1)])
            pltpu.sync_copy(dst_hbm.at[pl.ds(start + CH_BASE, 1)],
                            dst_v.at[pl.ds(CH_BASE, 1)])

        plsc.subcore_barrier()

        def gath(j, buf, sem):
            return pltpu.async_copy(y_hbm.at[src_v.at[j]], buf, sem)

        def gath_wait(j, buf, sem):
            pltpu.make_async_copy(y_hbm.at[src_v.at[j]], buf, sem).wait()

        gath(0, rows_a, semg_a)
        gath(1, rows_b, semg_b)

        def body(jj, _):
            j = jj * 2
            gath_wait(j, rows_a, semg_a)
            pltpu.sync_copy(rows_a, agg_sh.at[dst_v.at[j]], add=True)
            gath(j + 2, rows_a, semg_a)
            gath_wait(j + 1, rows_b, semg_b)
            pltpu.sync_copy(rows_b, agg_sh.at[dst_v.at[j + 1]], add=True)
            gath(j + 3, rows_b, semg_b)
            return 0

        PAIRS = (CH_BASE - 2) // 2
        lax.fori_loop(0, PAIRS, body, 0)
        ja = 2 * PAIRS
        gath_wait(ja, rows_a, semg_a)
        pltpu.sync_copy(rows_a, agg_sh.at[dst_v.at[ja]], add=True)
        gath_wait(ja + 1, rows_b, semg_b)
        pltpu.sync_copy(rows_b, agg_sh.at[dst_v.at[ja + 1]], add=True)
        for t in range(2 * PAIRS + 2, CH_BASE):
            pltpu.sync_copy(y_hbm.at[src_v.at[t]], rows_a)
            pltpu.sync_copy(rows_a, agg_sh.at[dst_v.at[t]], add=True)

        @pl.when(wid < CH_EXTRA)
        def _():
            pltpu.sync_copy(y_hbm.at[src_v.at[CH_BASE]], rows_b)
            pltpu.sync_copy(rows_b, agg_sh.at[dst_v.at[CH_BASE]], add=True)

        plsc.subcore_barrier()

        for off, sz in INIT_SLICES:
            sl = pl.ds(sid * ROWS_PER_TILE + off, sz)
            pltpu.sync_copy(agg_sh.at[sl], rows_a.at[pl.ds(0, sz)])
            pltpu.sync_copy(rows_a.at[pl.ds(0, sz)], out_hbm.at[cid, sl])

    return agg_kernel(y_ext, src2, dst2)


def _tc_scale(x, W, deg_col):
    BLK = 2000

    def body(x_ref, w_ref, d_ref, y_ref):
        dinv = lax.rsqrt(d_ref[...])
        xw = jnp.dot(x_ref[...], w_ref[...], preferred_element_type=jnp.float32)
        y_ref[...] = xw * dinv

    return pl.pallas_call(
        body,
        grid=(N // BLK,),
        in_specs=[
            pl.BlockSpec((BLK, F), lambda i: (i, 0)),
            pl.BlockSpec((F, C), lambda i: (0, 0)),
            pl.BlockSpec((BLK, 1), lambda i: (i, 0)),
        ],
        out_specs=pl.BlockSpec((BLK, C), lambda i: (i, 0)),
        out_shape=jax.ShapeDtypeStruct((N_PAD, C), jnp.float32),
    )(x, W, deg_col)


def _tc_final(aggp, deg_col, b2):
    BLK = 2000

    def body(a_ref, d_ref, b_ref, o_ref):
        dinv = lax.rsqrt(d_ref[...])
        o = (a_ref[0] + a_ref[1]) * dinv + b_ref[...]
        m = jnp.max(o, axis=1, keepdims=True)
        ex = jnp.exp(o - m)
        ssum = jnp.sum(ex, axis=1, keepdims=True)
        o_ref[...] = (o - m) - jnp.log(ssum)

    return pl.pallas_call(
        body,
        grid=(N // BLK,),
        in_specs=[
            pl.BlockSpec((NC, BLK, C), lambda i: (0, i, 0)),
            pl.BlockSpec((BLK, 1), lambda i: (i, 0)),
            pl.BlockSpec((1, C), lambda i: (0, 0)),
        ],
        out_specs=pl.BlockSpec((BLK, C), lambda i: (i, 0)),
        out_shape=jax.ShapeDtypeStruct((N, C), jnp.float32),
    )(aggp, deg_col, b2)


def kernel(x, edge_index, W, b):
    ei = edge_index.astype(jnp.int32)
    src2 = ei[0].reshape(NCHUNKS, CHUNK)
    dst2 = ei[1].reshape(NCHUNKS, CHUNK)
    src2, dst2 = lax.optimization_barrier((src2, dst2))
    degp = _sc_degree(dst2)
    deg_col = (degp[0, :N] + degp[1, :N] + 1.0).reshape(N, 1)
    y_ext = _tc_scale(x, W, deg_col)
    aggp = _sc_aggregate(y_ext, src2, dst2)
    return _tc_final(aggp, deg_col, b.reshape(1, C))


# speedup vs baseline: 1.0586x; 1.0586x over previous
import functools

import jax
import jax.numpy as jnp
from jax import lax
from jax.experimental import pallas as pl
from jax.experimental.pallas import tpu as pltpu
from jax.experimental.pallas import tpu_sc as plsc

N = 10000
E = 320000
F = 128
C = 64

NC = 2
NS = 16
NW = NC * NS
CHUNK = 512
NCHUNKS = E // CHUNK
CH_BASE = NCHUNKS // NW
CH_EXTRA = NCHUNKS % NW
N_PAD = 10240
ROWS_PER_TILE = N_PAD // NS
INIT_SLICES = []
_off = 0
while _off < ROWS_PER_TILE:
    _sz = min(CHUNK, ROWS_PER_TILE - _off)
    INIT_SLICES.append((_off, _sz))
    _off += _sz
ZCHUNK = 128


def _sc_mesh():
    return plsc.VectorSubcoreMesh(core_axis_name="c", subcore_axis_name="s")


_SC_PARAMS = pltpu.CompilerParams(use_tc_tiling_on_sc=False)


def _chunk_range(wid):
    start = CH_BASE * wid + lax.min(wid, CH_EXTRA)
    return start


def _sc_degree(dst2):
    @functools.partial(
        pl.kernel,
        out_type=jax.ShapeDtypeStruct((NC, N_PAD), jnp.float32),
        mesh=_sc_mesh(),
        compiler_params=_SC_PARAMS,
        scratch_types=[
            pltpu.VMEM((CH_BASE + 1, CHUNK), jnp.int32),
            pltpu.VMEM((CHUNK,), jnp.float32),
            pltpu.VMEM((ROWS_PER_TILE,), jnp.float32),
            pltpu.VMEM_SHARED((N_PAD,), jnp.float32),
        ],
    )
    def deg_kernel(ei_hbm, out_hbm, dst_v, ones_v, buf_v, deg_sh):
        cid = lax.axis_index("c")
        sid = lax.axis_index("s")
        wid = cid * NS + sid
        start = _chunk_range(wid)

        def fill(i, _):
            buf_v[pl.ds(i * 16, 16)] = jnp.zeros((16,), jnp.float32)
            return 0

        lax.fori_loop(0, ROWS_PER_TILE // 16, fill, 0)

        def fill1(i, _):
            ones_v[pl.ds(i * 16, 16)] = jnp.ones((16,), jnp.float32)
            return 0

        lax.fori_loop(0, CHUNK // 16, fill1, 0)

        tile_rows = pl.ds(sid * ROWS_PER_TILE, ROWS_PER_TILE)
        pltpu.sync_copy(buf_v, deg_sh.at[tile_rows])
        pltpu.sync_copy(ei_hbm.at[pl.ds(start, CH_BASE)],
                        dst_v.at[pl.ds(0, CH_BASE)])

        @pl.when(wid < CH_EXTRA)
        def _():
            pltpu.sync_copy(ei_hbm.at[pl.ds(start + CH_BASE, 1)],
                            dst_v.at[pl.ds(CH_BASE, 1)])

        plsc.subcore_barrier()

        def body(j, _):
            pltpu.sync_copy(ones_v, deg_sh.at[dst_v.at[j]], add=True)
            return 0

        lax.fori_loop(0, CH_BASE, body, 0)

        @pl.when(wid < CH_EXTRA)
        def _():
            pltpu.sync_copy(ones_v, deg_sh.at[dst_v.at[CH_BASE]], add=True)

        plsc.subcore_barrier()
        pltpu.sync_copy(deg_sh.at[tile_rows], buf_v)
        pltpu.sync_copy(buf_v, out_hbm.at[cid, tile_rows])

    return deg_kernel(dst2)


def _sc_aggregate(y_ext, src2, dst2):
    @functools.partial(
        pl.kernel,
        out_type=jax.ShapeDtypeStruct((NC, N_PAD, C), jnp.float32),
        mesh=_sc_mesh(),
        compiler_params=_SC_PARAMS,
        scratch_types=[
            pltpu.VMEM((CH_BASE + 1, CHUNK), jnp.int32),
            pltpu.VMEM((CH_BASE + 1, CHUNK), jnp.int32),
            pltpu.VMEM((CHUNK, C), jnp.float32),
            pltpu.VMEM((CHUNK, C), jnp.float32),
            pltpu.VMEM_SHARED((N_PAD, C), jnp.float32),
            pltpu.SemaphoreType.DMA,
            pltpu.SemaphoreType.DMA,
        ],
    )
    def agg_kernel(y_hbm, src_hbm, dst_hbm, out_hbm, src_v, dst_v, rows_a,
                   rows_b, agg_sh, semg_a, semg_b):
        cid = lax.axis_index("c")
        sid = lax.axis_index("s")
        wid = cid * NS + sid
        start = _chunk_range(wid)

        def zero_row(i, _):
            for k in range(C // 16):
                rows_b[i, pl.ds(k * 16, 16)] = jnp.zeros((16,), jnp.float32)
            return 0

        lax.fori_loop(0, CHUNK, zero_row, 0)

        for off, sz in INIT_SLICES:
            sl = pl.ds(sid * ROWS_PER_TILE + off, sz)

            @pl.when(cid == 0)
            def _():
                pltpu.sync_copy(y_hbm.at[sl], rows_a.at[pl.ds(0, sz)])
                pltpu.sync_copy(rows_a.at[pl.ds(0, sz)], agg_sh.at[sl])

            @pl.when(cid != 0)
            def _():
                pltpu.sync_copy(rows_b.at[pl.ds(0, sz)], agg_sh.at[sl])

        pltpu.sync_copy(src_hbm.at[pl.ds(start, CH_BASE)],
                        src_v.at[pl.ds(0, CH_BASE)])
        pltpu.sync_copy(dst_hbm.at[pl.ds(start, CH_BASE)],
                        dst_v.at[pl.ds(0, CH_BASE)])

        @pl.when(wid < CH_EXTRA)
        def _():
            pltpu.sync_copy(src_hbm.at[pl.ds(start + CH_BASE, 1)],
                            src_v.at[pl.ds(CH_BASE, 1)])
            pltpu.sync_copy(dst_hbm.at[pl.ds(start + CH_BASE, 1)],
                            dst_v.at[pl.ds(CH_BASE, 1)])

        plsc.subcore_barrier()

        def gath(j, buf, sem):
            return pltpu.async_copy(y_hbm.at[src_v.at[j]], buf, sem)

        def gath_wait(j, buf, sem):
            pltpu.make_async_copy(y_hbm.at[src_v.at[j]], buf, sem).wait()

        gath(0, rows_a, semg_a)
        gath(1, rows_b, semg_b)

        def body(jj, _):
            j = jj * 2
            gath_wait(j, rows_a, semg_a)
            pltpu.sync_copy(rows_a, agg_sh.at[dst_v.at[j]], add=True)
            gath(j + 2, rows_a, semg_a)
            gath_wait(j + 1, rows_b, semg_b)
            pltpu.sync_copy(rows_b, agg_sh.at[dst_v.at[j + 1]], add=True)
            gath(j + 3, rows_b, semg_b)
            return 0

        PAIRS = (CH_BASE - 2) // 2
        lax.fori_loop(0, PAIRS, body, 0)
        ja = 2 * PAIRS
        gath_wait(ja, rows_a, semg_a)
        pltpu.sync_copy(rows_a, agg_sh.at[dst_v.at[ja]], add=True)
        gath_wait(ja + 1, rows_b, semg_b)
        pltpu.sync_copy(rows_b, agg_sh.at[dst_v.at[ja + 1]], add=True)
        for t in range(2 * PAIRS + 2, CH_BASE):
            pltpu.sync_copy(y_hbm.at[src_v.at[t]], rows_a)
            pltpu.sync_copy(rows_a, agg_sh.at[dst_v.at[t]], add=True)

        @pl.when(wid < CH_EXTRA)
        def _():
            pltpu.sync_copy(y_hbm.at[src_v.at[CH_BASE]], rows_b)
            pltpu.sync_copy(rows_b, agg_sh.at[dst_v.at[CH_BASE]], add=True)

        plsc.subcore_barrier()

        for off, sz in INIT_SLICES:
            sl = pl.ds(sid * ROWS_PER_TILE + off, sz)
            pltpu.sync_copy(agg_sh.at[sl], rows_a.at[pl.ds(0, sz)])
            pltpu.sync_copy(rows_a.at[pl.ds(0, sz)], out_hbm.at[cid, sl])

    return agg_kernel(y_ext, src2, dst2)


def _tc_scale(x, W, deg_col):
    BLK = 2000

    def body(x_ref, w_ref, d_ref, y_ref):
        dinv = lax.rsqrt(d_ref[...])
        xw = jnp.dot(x_ref[...], w_ref[...], preferred_element_type=jnp.float32)
        y_ref[...] = xw * dinv

    return pl.pallas_call(
        body,
        grid=(N // BLK,),
        in_specs=[
            pl.BlockSpec((BLK, F), lambda i: (i, 0)),
            pl.BlockSpec((F, C), lambda i: (0, 0)),
            pl.BlockSpec((BLK, 1), lambda i: (i, 0)),
        ],
        out_specs=pl.BlockSpec((BLK, C), lambda i: (i, 0)),
        out_shape=jax.ShapeDtypeStruct((N_PAD, C), jnp.float32),
    )(x, W, deg_col)


def _tc_final(agg128, deg2, b128):
    BLK2 = 1000
    BLK = 2 * BLK2

    def body(a_ref, d_ref, b_ref, o_ref):
        a = a_ref[0] + a_ref[1]
        d2 = d_ref[...]
        dinv = lax.rsqrt(d2)
        dinv128 = jnp.concatenate(
            [jnp.broadcast_to(dinv[:, 0:1], (BLK2, C)),
             jnp.broadcast_to(dinv[:, 1:2], (BLK2, C))], axis=1)
        o = a * dinv128 + b_ref[...]
        left = o[:, :C]
        right = o[:, C:]

        def lsm(v):
            m = jnp.max(v, axis=1, keepdims=True)
            ex = jnp.exp(v - m)
            return (v - m) - jnp.log(jnp.sum(ex, axis=1, keepdims=True))

        o_ref[...] = jnp.concatenate([lsm(left), lsm(right)], axis=1)

    out128 = pl.pallas_call(
        body,
        grid=(N // BLK,),
        in_specs=[
            pl.BlockSpec((NC, BLK2, 2 * C), lambda i: (0, i, 0)),
            pl.BlockSpec((BLK2, 2), lambda i: (i, 0)),
            pl.BlockSpec((1, 2 * C), lambda i: (0, 0)),
        ],
        out_specs=pl.BlockSpec((BLK2, 2 * C), lambda i: (i, 0)),
        out_shape=jax.ShapeDtypeStruct((N // 2, 2 * C), jnp.float32),
    )(agg128, deg2, b128)
    return out128.reshape(N, C)


def kernel(x, edge_index, W, b):
    ei = edge_index.astype(jnp.int32)
    src2 = ei[0].reshape(NCHUNKS, CHUNK)
    dst2 = ei[1].reshape(NCHUNKS, CHUNK)
    src2, dst2 = lax.optimization_barrier((src2, dst2))
    degp = _sc_degree(dst2)
    degsum = degp[0] + degp[1] + 1.0
    deg_col = degsum[:N].reshape(N, 1)
    y_ext = _tc_scale(x, W, deg_col)
    aggp = _sc_aggregate(y_ext, src2, dst2)
    agg128 = aggp.reshape(NC, N_PAD // 2, 2 * C)
    deg2 = degsum.reshape(N_PAD // 2, 2)
    b128 = jnp.concatenate([b, b]).reshape(1, 2 * C)
    return _tc_final(agg128, deg2, b128)
